# Initial kernel scaffold; baseline (speedup 1.0000x reference)
import functools

import jax
import jax.numpy as jnp
from jax import lax
from jax.experimental import pallas as pl
from jax.experimental.pallas import tpu as pltpu
from jax.experimental.pallas import tpu_sc as plsc

_D = 128
_N = 10000
_M = 5000
_MP = 5120
_NP = 10240
_CH = 128
_BN_EPS = 1e-5
_E_COEF = 0.1


def _tc_prenorm(X, W, b, gamma, beta):
    def body(x_ref, w_ref, b_ref, g_ref, be_ref, o_ref):
        H = lax.dot_general(x_ref[...], w_ref[...], (((1,), (1,)), ((), ())),
                            preferred_element_type=jnp.float32)
        H = H + b_ref[...]
        mean = jnp.mean(H, axis=0, keepdims=True)
        var = jnp.mean((H - mean) ** 2, axis=0, keepdims=True)
        o_ref[...] = (H - mean) * lax.rsqrt(var + _BN_EPS) * g_ref[...] + be_ref[...]

    return pl.pallas_call(
        body,
        out_shape=jax.ShapeDtypeStruct((_N, _D), jnp.float32),
    )(X, W, b.reshape(1, _D), gamma.reshape(1, _D), beta.reshape(1, _D))


def _sc_v2e(mesh, nch, hn, pv, pe, zeros_hbm, ones_hbm):
    nc, ns = mesh.num_cores, mesh.num_subcores
    nw = nc * ns
    rps = _MP // ns

    @functools.partial(
        pl.kernel,
        mesh=mesh,
        out_type=(jax.ShapeDtypeStruct((nc * 3 * _MP, _D), jnp.float32),
                  jax.ShapeDtypeStruct((nc * 3 * _MP, _D), jnp.float32)),
        scratch_types=[
            pltpu.VMEM((_CH,), jnp.int32),
            pltpu.VMEM((_CH,), jnp.int32),
            pltpu.VMEM((_CH, _D), jnp.float32),
            pltpu.VMEM((_CH, _D), jnp.float32),
            pltpu.VMEM_SHARED((_MP, _D), jnp.float32),
            pltpu.VMEM_SHARED((_MP, _D), jnp.float32),
            pltpu.SemaphoreType.DMA,
        ],
    )
    def k(hn_hbm, pv_hbm, pe_hbm, z_hbm, one_hbm, osum, ocnt,
          idxv, idxe, rows, ones_v, accs, accc, sem):
        cid = lax.axis_index("c")
        sid = lax.axis_index("s")
        wid = sid * nc + cid
        pltpu.sync_copy(one_hbm, ones_v)
        for g in range(3):
            pltpu.sync_copy(z_hbm.at[pl.ds(sid * rps, rps)],
                            accs.at[pl.ds(sid * rps, rps)])
            pltpu.sync_copy(z_hbm.at[pl.ds(sid * rps, rps)],
                            accc.at[pl.ds(sid * rps, rps)])
            plsc.subcore_barrier()

            @pl.loop(0, nch)
            def _(ch):
                r = (g * nw + wid) * nch + ch
                pltpu.sync_copy(pv_hbm.at[r], idxv)
                pltpu.sync_copy(pe_hbm.at[r], idxe)
                pltpu.async_copy(hn_hbm.at[idxv], rows, sem).wait()
                pltpu.sync_copy(rows, accs.at[idxe], add=True)
                pltpu.sync_copy(ones_v, accc.at[idxe], add=True)

            plsc.subcore_barrier()
            base = (cid * 3 + g) * _MP + sid * rps
            pltpu.sync_copy(accs.at[pl.ds(sid * rps, rps)],
                            osum.at[pl.ds(base, rps)])
            pltpu.sync_copy(accc.at[pl.ds(sid * rps, rps)],
                            ocnt.at[pl.ds(base, rps)])
            plsc.subcore_barrier()

    return k(hn, pv, pe, zeros_hbm, ones_hbm)


def _sc_e2v(mesh, nch, ys, pv, pe, zeros_hbm):
    nc, ns = mesh.num_cores, mesh.num_subcores
    rps = _NP // ns

    @functools.partial(
        pl.kernel,
        mesh=mesh,
        out_type=jax.ShapeDtypeStruct((nc * _NP, _D), jnp.float32),
        scratch_types=[
            pltpu.VMEM((_CH,), jnp.int32),
            pltpu.VMEM((_CH,), jnp.int32),
            pltpu.VMEM((_CH, _D), jnp.float32),
            pltpu.VMEM_SHARED((_NP, _D), jnp.float32),
            pltpu.SemaphoreType.DMA,
        ],
    )
    def k(ys_hbm, pv_hbm, pe_hbm, z_hbm, out, idxv, idxe, rows, acc, sem):
        cid = lax.axis_index("c")
        sid = lax.axis_index("s")
        wid = sid * nc + cid
        pltpu.sync_copy(z_hbm.at[pl.ds(sid * rps, rps)],
                        acc.at[pl.ds(sid * rps, rps)])
        plsc.subcore_barrier()

        @pl.loop(0, nch)
        def _(ch):
            r = wid * nch + ch
            pltpu.sync_copy(pv_hbm.at[r], idxv)
            pltpu.sync_copy(pe_hbm.at[r], idxe)
            pltpu.async_copy(ys_hbm.at[idxe], rows, sem).wait()
            pltpu.sync_copy(rows, acc.at[idxv], add=True)

        plsc.subcore_barrier()
        base = cid * _NP + sid * rps
        pltpu.sync_copy(acc.at[pl.ds(sid * rps, rps)], out.at[pl.ds(base, rps)])

    return k(ys, pv, pe, zeros_hbm)


def _tc_gate(nc, sums_p, cnt_p, w_gate, a_vec):
    def body(sp_ref, cp_ref, wg_ref, a_ref, ys_ref):
        s = sp_ref[0, 0]
        c = cp_ref[0, 0]
        for i in range(1, nc):
            s = s + sp_ref[i, 0]
            c = c + cp_ref[i, 0]
        Y = s / jnp.maximum(c, 1.0)
        alpha = jnp.tanh(jnp.sum(Y * wg_ref[...], axis=1, keepdims=True))
        sc = jnp.where(alpha >= 0.0, alpha, 0.2 * alpha)
        sc = jnp.clip(sc, 0.0, 5.0) * a_ref[0, 0]
        ys_ref[0] = sc * Y

    return pl.pallas_call(
        body,
        grid=(3,),
        in_specs=[
            pl.BlockSpec((nc, 1, _MP, _D), lambda g: (0, g, 0, 0)),
            pl.BlockSpec((nc, 1, _MP, _D), lambda g: (0, g, 0, 0)),
            pl.BlockSpec((1, _D), lambda g: (0, 0)),
            pl.BlockSpec((1, 1), lambda g: (g, 0)),
        ],
        out_specs=pl.BlockSpec((1, _MP, _D), lambda g: (g, 0, 0)),
        out_shape=jax.ShapeDtypeStruct((3, _MP, _D), jnp.float32),
    )(sums_p, cnt_p, w_gate.reshape(1, _D), a_vec)


def _tc_combine(nc, node_p, hn):
    def body(np_ref, hn_ref, o_ref):
        t = np_ref[0, :_N]
        for i in range(1, nc):
            t = t + np_ref[i, :_N]
        t = t + (3.0 * _E_COEF) * hn_ref[...]
        o_ref[...] = jnp.where(t >= 0.0, t, 0.01 * t)

    return pl.pallas_call(
        body,
        out_shape=jax.ShapeDtypeStruct((_N, _D), jnp.float32),
    )(node_p, hn)


def _pad_reshape(x, tot, fill):
    x = x.astype(jnp.int32)
    pad = tot - x.shape[0]
    if pad:
        x = jnp.concatenate([x, jnp.full((pad,), fill, jnp.int32)])
    return x


def kernel(X, hier_pair_v, hier_pair_e, cooc_pair_v, cooc_pair_e,
           cit_pair_v, cit_pair_e, W_theta, b_theta, gamma, beta,
           w_gate, a1, a2, a3):
    mesh = plsc.VectorSubcoreMesh(core_axis_name="c", subcore_axis_name="s")
    nc, ns = mesh.num_cores, mesh.num_subcores
    nw = nc * ns
    gsz = nw * _CH

    hn = _tc_prenorm(X, W_theta, b_theta, gamma, beta)

    zeros_hbm = jnp.zeros((_NP, _D), jnp.float32)
    ones_hbm = jnp.ones((_CH, _D), jnp.float32)

    npairs = hier_pair_v.shape[0]
    nch1 = -(-npairs // gsz)
    tot1 = nch1 * gsz
    pv1 = jnp.stack([_pad_reshape(v, tot1, 0)
                     for v in (hier_pair_v, cooc_pair_v, cit_pair_v)])
    pe1 = jnp.stack([_pad_reshape(e, tot1, _M)
                     for e in (hier_pair_e, cooc_pair_e, cit_pair_e)])
    pv1 = pv1.reshape(3 * nw * nch1, _CH)
    pe1 = pe1.reshape(3 * nw * nch1, _CH)

    sums_p, cnt_p = _sc_v2e(mesh, nch1, hn, pv1, pe1, zeros_hbm, ones_hbm)
    sums_p = sums_p.reshape(nc, 3, _MP, _D)
    cnt_p = cnt_p.reshape(nc, 3, _MP, _D)

    a_vec = jnp.stack([a1, a2, a3]).reshape(3, 1)
    ys = _tc_gate(nc, sums_p, cnt_p, w_gate, a_vec)
    ys_flat = ys.reshape(3 * _MP, _D)

    v_all = jnp.concatenate([hier_pair_v.astype(jnp.int32),
                             cooc_pair_v.astype(jnp.int32),
                             cit_pair_v.astype(jnp.int32)])
    e_all = jnp.concatenate([hier_pair_e.astype(jnp.int32),
                             cooc_pair_e.astype(jnp.int32) + _MP,
                             cit_pair_e.astype(jnp.int32) + 2 * _MP])
    nch2 = -(-v_all.shape[0] // gsz)
    tot2 = nch2 * gsz
    pv2 = _pad_reshape(v_all, tot2, _N).reshape(nw * nch2, _CH)
    pe2 = _pad_reshape(e_all, tot2, 0).reshape(nw * nch2, _CH)

    node_p = _sc_e2v(mesh, nch2, ys_flat, pv2, pe2, zeros_hbm)
    node_p = node_p.reshape(nc, _NP, _D)

    return _tc_combine(nc, node_p, hn)


# speedup vs baseline: 6.7542x; 6.7542x over previous
import functools

import jax
import jax.numpy as jnp
from jax import lax
from jax.experimental import pallas as pl
from jax.experimental.pallas import tpu as pltpu
from jax.experimental.pallas import tpu_sc as plsc

_D = 128
_N = 10000
_M = 5000
_MP = 5120
_NP = 10240
_CH = 128
_BN_EPS = 1e-5
_E_COEF = 0.1


def _tc_prenorm(X, W, b, gamma, beta):
    def body(x_ref, w_ref, b_ref, g_ref, be_ref, o_ref):
        H = lax.dot_general(x_ref[...], w_ref[...], (((1,), (1,)), ((), ())),
                            preferred_element_type=jnp.float32)
        H = H + b_ref[...]
        mean = jnp.mean(H, axis=0, keepdims=True)
        var = jnp.mean((H - mean) ** 2, axis=0, keepdims=True)
        o_ref[...] = (H - mean) * lax.rsqrt(var + _BN_EPS) * g_ref[...] + be_ref[...]

    return pl.pallas_call(
        body,
        out_shape=jax.ShapeDtypeStruct((_N, _D), jnp.float32),
    )(X, W, b.reshape(1, _D), gamma.reshape(1, _D), beta.reshape(1, _D))


def _sc_v2e(mesh, nch, hn, pv, pe, zeros_hbm, ones_hbm):
    nc, ns = mesh.num_cores, mesh.num_subcores
    nw = nc * ns
    rps = _MP // ns

    @functools.partial(
        pl.kernel,
        mesh=mesh,
        out_type=(jax.ShapeDtypeStruct((nc * 3 * _MP, _D), jnp.float32),
                  jax.ShapeDtypeStruct((nc * 3 * _MP, _D), jnp.float32)),
        scratch_types=[
            pltpu.VMEM((_CH,), jnp.int32),
            pltpu.VMEM((_CH,), jnp.int32),
            pltpu.VMEM((_CH, _D), jnp.float32),
            pltpu.VMEM((_CH, _D), jnp.float32),
            pltpu.VMEM_SHARED((_MP, _D), jnp.float32),
            pltpu.VMEM_SHARED((_MP, _D), jnp.float32),
            pltpu.SemaphoreType.DMA,
        ],
    )
    def k(hn_hbm, pv_hbm, pe_hbm, z_hbm, one_hbm, osum, ocnt,
          idxv, idxe, rows, ones_v, accs, accc, sem):
        cid = lax.axis_index("c")
        sid = lax.axis_index("s")
        wid = sid * nc + cid
        pltpu.sync_copy(one_hbm, ones_v)
        for g in range(3):
            pltpu.sync_copy(z_hbm.at[pl.ds(sid * rps, rps)],
                            accs.at[pl.ds(sid * rps, rps)])
            pltpu.sync_copy(z_hbm.at[pl.ds(sid * rps, rps)],
                            accc.at[pl.ds(sid * rps, rps)])
            plsc.subcore_barrier()

            @pl.loop(0, nch)
            def _(ch):
                r = (g * nw + wid) * nch + ch
                pltpu.sync_copy(pv_hbm.at[r], idxv)
                pltpu.sync_copy(pe_hbm.at[r], idxe)
                pltpu.async_copy(hn_hbm.at[idxv], rows, sem).wait()
                pltpu.sync_copy(rows, accs.at[idxe], add=True)
                pltpu.sync_copy(ones_v, accc.at[idxe], add=True)

            plsc.subcore_barrier()
            base = (cid * 3 + g) * _MP + sid * rps
            pltpu.sync_copy(accs.at[pl.ds(sid * rps, rps)],
                            osum.at[pl.ds(base, rps)])
            pltpu.sync_copy(accc.at[pl.ds(sid * rps, rps)],
                            ocnt.at[pl.ds(base, rps)])
            plsc.subcore_barrier()

    return k(hn, pv, pe, zeros_hbm, ones_hbm)


def _sc_e2v(mesh, nch, ys, pv, pe, zeros_hbm):
    nc, ns = mesh.num_cores, mesh.num_subcores
    rps = _NP // ns

    @functools.partial(
        pl.kernel,
        mesh=mesh,
        out_type=jax.ShapeDtypeStruct((nc * _NP, _D), jnp.float32),
        scratch_types=[
            pltpu.VMEM((_CH,), jnp.int32),
            pltpu.VMEM((_CH,), jnp.int32),
            pltpu.VMEM((_CH, _D), jnp.float32),
            pltpu.VMEM_SHARED((_NP, _D), jnp.float32),
            pltpu.SemaphoreType.DMA,
        ],
    )
    def k(ys_hbm, pv_hbm, pe_hbm, z_hbm, out, idxv, idxe, rows, acc, sem):
        cid = lax.axis_index("c")
        sid = lax.axis_index("s")
        wid = sid * nc + cid
        pltpu.sync_copy(z_hbm.at[pl.ds(sid * rps, rps)],
                        acc.at[pl.ds(sid * rps, rps)])
        plsc.subcore_barrier()

        @pl.loop(0, nch)
        def _(ch):
            r = wid * nch + ch
            pltpu.sync_copy(pv_hbm.at[r], idxv)
            pltpu.sync_copy(pe_hbm.at[r], idxe)
            pltpu.async_copy(ys_hbm.at[idxe], rows, sem).wait()
            pltpu.sync_copy(rows, acc.at[idxv], add=True)

        plsc.subcore_barrier()
        base = cid * _NP + sid * rps
        pltpu.sync_copy(acc.at[pl.ds(sid * rps, rps)], out.at[pl.ds(base, rps)])

    return k(ys, pv, pe, zeros_hbm)


def _tc_gate(nc, sums_p, cnt_p, w_gate, a_vec):
    def body(sp_ref, cp_ref, wg_ref, a_ref, ys_ref):
        s = sp_ref[0, 0]
        c = cp_ref[0, 0]
        for i in range(1, nc):
            s = s + sp_ref[i, 0]
            c = c + cp_ref[i, 0]
        Y = s / jnp.maximum(c, 1.0)
        alpha = jnp.tanh(jnp.sum(Y * wg_ref[...], axis=1, keepdims=True))
        sc = jnp.where(alpha >= 0.0, alpha, 0.2 * alpha)
        sc = jnp.clip(sc, 0.0, 5.0) * a_ref[0, 0, 0]
        ys_ref[0] = sc * Y

    return pl.pallas_call(
        body,
        grid=(3,),
        in_specs=[
            pl.BlockSpec((nc, 1, _MP, _D), lambda g: (0, g, 0, 0)),
            pl.BlockSpec((nc, 1, _MP, _D), lambda g: (0, g, 0, 0)),
            pl.BlockSpec((1, _D), lambda g: (0, 0)),
            pl.BlockSpec((1, 1, 1), lambda g: (g, 0, 0)),
        ],
        out_specs=pl.BlockSpec((1, _MP, _D), lambda g: (g, 0, 0)),
        out_shape=jax.ShapeDtypeStruct((3, _MP, _D), jnp.float32),
    )(sums_p, cnt_p, w_gate.reshape(1, _D), a_vec)


def _tc_combine(nc, node_p, hn):
    def body(np_ref, hn_ref, o_ref):
        t = np_ref[0, :_N]
        for i in range(1, nc):
            t = t + np_ref[i, :_N]
        t = t + (3.0 * _E_COEF) * hn_ref[...]
        o_ref[...] = jnp.where(t >= 0.0, t, 0.01 * t)

    return pl.pallas_call(
        body,
        out_shape=jax.ShapeDtypeStruct((_N, _D), jnp.float32),
    )(node_p, hn)


def _pad_reshape(x, tot, fill):
    x = x.astype(jnp.int32)
    pad = tot - x.shape[0]
    if pad:
        x = jnp.concatenate([x, jnp.full((pad,), fill, jnp.int32)])
    return x


def kernel(X, hier_pair_v, hier_pair_e, cooc_pair_v, cooc_pair_e,
           cit_pair_v, cit_pair_e, W_theta, b_theta, gamma, beta,
           w_gate, a1, a2, a3):
    mesh = plsc.VectorSubcoreMesh(core_axis_name="c", subcore_axis_name="s")
    nc, ns = mesh.num_cores, mesh.num_subcores
    nw = nc * ns
    gsz = nw * _CH

    hn = _tc_prenorm(X, W_theta, b_theta, gamma, beta)

    zeros_hbm = jnp.zeros((_NP, _D), jnp.float32)
    ones_hbm = jnp.ones((_CH, _D), jnp.float32)

    npairs = hier_pair_v.shape[0]
    nch1 = -(-npairs // gsz)
    tot1 = nch1 * gsz
    pv1 = jnp.stack([_pad_reshape(v, tot1, 0)
                     for v in (hier_pair_v, cooc_pair_v, cit_pair_v)])
    pe1 = jnp.stack([_pad_reshape(e, tot1, _M)
                     for e in (hier_pair_e, cooc_pair_e, cit_pair_e)])
    pv1 = pv1.reshape(3 * nw * nch1, _CH)
    pe1 = pe1.reshape(3 * nw * nch1, _CH)

    sums_p, cnt_p = _sc_v2e(mesh, nch1, hn, pv1, pe1, zeros_hbm, ones_hbm)
    sums_p = sums_p.reshape(nc, 3, _MP, _D)
    cnt_p = cnt_p.reshape(nc, 3, _MP, _D)

    a_vec = jnp.stack([a1, a2, a3]).reshape(3, 1, 1)
    ys = _tc_gate(nc, sums_p, cnt_p, w_gate, a_vec)
    ys_flat = ys.reshape(3 * _MP, _D)

    v_all = jnp.concatenate([hier_pair_v.astype(jnp.int32),
                             cooc_pair_v.astype(jnp.int32),
                             cit_pair_v.astype(jnp.int32)])
    e_all = jnp.concatenate([hier_pair_e.astype(jnp.int32),
                             cooc_pair_e.astype(jnp.int32) + _MP,
                             cit_pair_e.astype(jnp.int32) + 2 * _MP])
    nch2 = -(-v_all.shape[0] // gsz)
    tot2 = nch2 * gsz
    pv2 = _pad_reshape(v_all, tot2, _N).reshape(nw * nch2, _CH)
    pe2 = _pad_reshape(e_all, tot2, 0).reshape(nw * nch2, _CH)

    node_p = _sc_e2v(mesh, nch2, ys_flat, pv2, pe2, zeros_hbm)
    node_p = node_p.reshape(nc, _NP, _D)

    return _tc_combine(nc, node_p, hn)
